# Initial kernel scaffold; baseline (speedup 1.0000x reference)
import jax
import jax.numpy as jnp
from jax.experimental import pallas as pl


def kernel(x, mask, projection, codebooks):
    raise NotImplementedError("write your pallas kernel here")


# speedup vs baseline: 1.0528x; 1.0528x over previous
import jax
import jax.numpy as jnp
from jax.experimental import pallas as pl

_ROW_BLOCK = 256
_PROJ_BLOCK = 2048
_ARGMIN_CHUNK = 2816


def _proj_kernel(x_ref, proj_ref, xp_ref):
    xp_ref[...] = jax.lax.dot_general(
        x_ref[...], proj_ref[...], (((1,), (0,)), ((), ())),
        preferred_element_type=jnp.float32)


def _vq_kernel(xp_ref, mx_ref, xpow_ref, cn_ref, cpow_ref, mask_ref, out_ref):
    xn = xp_ref[...] / mx_ref[...]
    xc = jax.lax.dot_general(
        xn, cn_ref[...], (((1,), (1,)), ((), ())),
        preferred_element_type=jnp.float32)
    d = jnp.sqrt((xpow_ref[...] + cpow_ref[...]) - 2.0 * xc)

    k = d.shape[1]
    rb = d.shape[0]
    best_v = jnp.full((rb, 1), jnp.inf, jnp.float32)
    best_i = jnp.zeros((rb, 1), jnp.int32)
    c0 = 0
    while c0 < k:
        w = min(_ARGMIN_CHUNK, k - c0)
        seg = d[:, c0:c0 + w]
        v = jnp.min(seg, axis=1, keepdims=True)
        iota = jax.lax.broadcasted_iota(jnp.int32, seg.shape, 1)
        i = jnp.min(jnp.where(seg == v, iota, k), axis=1, keepdims=True) + c0
        upd = v < best_v
        best_v = jnp.where(upd, v.astype(jnp.bfloat16).astype(jnp.float32),
                           best_v)
        best_i = jnp.where(upd, i, best_i)
        c0 += w
    out_ref[...] = jnp.where(mask_ref[...] != 0, best_i, 0)


def kernel(x, mask, projection, codebooks):
    b, t, dim = x.shape
    k = codebooks.shape[0]
    n = b * t
    x2 = x.reshape(n, dim)

    xp = pl.pallas_call(
        _proj_kernel,
        grid=(n // _PROJ_BLOCK,),
        in_specs=[
            pl.BlockSpec((_PROJ_BLOCK, dim), lambda i: (i, 0)),
            pl.BlockSpec((dim, dim), lambda i: (0, 0)),
        ],
        out_specs=pl.BlockSpec((_PROJ_BLOCK, dim), lambda i: (i, 0)),
        out_shape=jax.ShapeDtypeStruct((n, dim), jnp.float32),
    )(x2, projection)

    m_x = jnp.maximum(jnp.sqrt(jnp.sum(xp * xp, axis=-1, keepdims=True)), 1e-12)
    x_pow = jnp.sum((xp / m_x) ** 2, axis=-1, keepdims=True)
    m_c = jnp.maximum(jnp.sqrt(jnp.sum(codebooks * codebooks, axis=-1,
                                       keepdims=True)), 1e-12)
    cn = codebooks / m_c
    c_pow = jnp.sum(cn * cn, axis=-1).reshape(1, k)

    m2 = mask.reshape(n, 1).astype(jnp.int32)

    out = pl.pallas_call(
        _vq_kernel,
        grid=(n // _ROW_BLOCK,),
        in_specs=[
            pl.BlockSpec((_ROW_BLOCK, dim), lambda i: (i, 0)),
            pl.BlockSpec((_ROW_BLOCK, 1), lambda i: (i, 0)),
            pl.BlockSpec((_ROW_BLOCK, 1), lambda i: (i, 0)),
            pl.BlockSpec((k, dim), lambda i: (0, 0)),
            pl.BlockSpec((1, k), lambda i: (0, 0)),
            pl.BlockSpec((_ROW_BLOCK, 1), lambda i: (i, 0)),
        ],
        out_specs=pl.BlockSpec((_ROW_BLOCK, 1), lambda i: (i, 0)),
        out_shape=jax.ShapeDtypeStruct((n, 1), jnp.int32),
    )(xp, m_x, x_pow, cn, c_pow, m2)

    return out.reshape(b, t)
